# baseline (speedup 1.0000x reference)
import functools

import jax
import jax.numpy as jnp
from jax import lax
from jax.experimental import pallas as pl
from jax.experimental.pallas import tpu as pltpu
from jax.experimental.pallas import tpu_sc as plsc

NUM_CORES = 2
NUM_SUBCORES = 16
NUM_WORKERS = NUM_CORES * NUM_SUBCORES

ROWS_PER_GATHER = 128
GATHERS_PER_CHUNK = 8
CHUNK = ROWS_PER_GATHER * GATHERS_PER_CHUNK


@functools.lru_cache(maxsize=None)
def _make_kernel(B, V, D):
    assert B % (NUM_WORKERS * CHUNK) == 0
    b_per_w = B // NUM_WORKERS
    nchunks = b_per_w // CHUNK
    groups_pw = b_per_w // ROWS_PER_GATHER

    mesh = plsc.VectorSubcoreMesh(core_axis_name="c", subcore_axis_name="s")

    @functools.partial(
        pl.kernel,
        mesh=mesh,
        compiler_params=pltpu.CompilerParams(use_tc_tiling_on_sc=False),
        out_type=jax.ShapeDtypeStruct((B, D), jnp.float32),
        scratch_types=[
            pltpu.VMEM((groups_pw, ROWS_PER_GATHER), jnp.int32),
            pltpu.VMEM((CHUNK, D), jnp.float32),
            pltpu.SemaphoreType.DMA,
        ],
    )
    def emb(table_hbm, idx_hbm, out_hbm, idx_v, rows_v, gsem):
        wid = lax.axis_index("s") * NUM_CORES + lax.axis_index("c")
        gbase = wid * groups_pw
        rbase = wid * b_per_w
        pltpu.sync_copy(idx_hbm.at[pl.ds(gbase, groups_pw)], idx_v)

        def body(g, carry):
            descs = []
            for j in range(GATHERS_PER_CHUNK):
                d = pltpu.async_copy(
                    table_hbm.at[idx_v.at[g * GATHERS_PER_CHUNK + j]],
                    rows_v.at[pl.ds(j * ROWS_PER_GATHER, ROWS_PER_GATHER)],
                    gsem,
                )
                descs.append(d)
            for d in descs:
                d.wait()
            pltpu.sync_copy(rows_v, out_hbm.at[pl.ds(rbase + g * CHUNK, CHUNK)])
            return carry

        lax.fori_loop(0, nchunks, body, 0)

    return emb


def kernel(x, weight):
    B = x.size
    D = weight.shape[1]
    idx = x.reshape(B // ROWS_PER_GATHER, ROWS_PER_GATHER).astype(jnp.int32)
    emb = _make_kernel(B, weight.shape[0], D)
    out = emb(weight, idx)
    return out.reshape(B, 1, D)


# speedup vs baseline: 1.3577x; 1.3577x over previous
import functools

import jax
import jax.numpy as jnp
from jax import lax
from jax.experimental import pallas as pl
from jax.experimental.pallas import tpu as pltpu
from jax.experimental.pallas import tpu_sc as plsc

NUM_CORES = 2
NUM_SUBCORES = 16
NUM_WORKERS = NUM_CORES * NUM_SUBCORES

LANES = 16
G = 128


WPAD = 128


@functools.lru_cache(maxsize=None)
def _make_kernel(B, V, D):
    assert D == 32 and B % (NUM_WORKERS * 2 * G) == 0
    b_per_w = B // NUM_WORKERS
    groups_pw = b_per_w // G
    pairs = groups_pw // 2
    tiles_f = D // 8

    mesh = plsc.VectorSubcoreMesh(core_axis_name="c", subcore_axis_name="s")

    @functools.partial(
        pl.kernel,
        mesh=mesh,
        compiler_params=pltpu.CompilerParams(use_tc_tiling_on_sc=False,
                                             needs_layout_passes=False),
        out_type=jax.ShapeDtypeStruct((tiles_f, B // G, 8, G), jnp.float32),
        scratch_types=[
            pltpu.VMEM((groups_pw, G), jnp.int32),
            pltpu.VMEM((G, WPAD), jnp.float32),
            pltpu.VMEM((G, WPAD), jnp.float32),
            pltpu.VMEM((D, G), jnp.float32),
            pltpu.VMEM((D, G), jnp.float32),
            pltpu.SemaphoreType.DMA,
            pltpu.SemaphoreType.DMA,
            pltpu.SemaphoreType.DMA,
            pltpu.SemaphoreType.DMA,
        ],
    )
    def emb(table_hbm, idx_hbm, out_hbm, idx_v, rows0, rows1, tb0, tb1,
            gsem0, gsem1, osem0, osem1):
        wid = lax.axis_index("s") * NUM_CORES + lax.axis_index("c")
        gbase = wid * groups_pw
        pltpu.sync_copy(idx_hbm.at[pl.ds(gbase, groups_pw)], idx_v)

        def gather(g, rows, gsem):
            return pltpu.make_async_copy(table_hbm.at[idx_v.at[g]], rows, gsem)

        def transpose(rows, tb):
            iot = lax.iota(jnp.int32, 16)

            def blk(bi, c):
                irow = bi * LANES + iot
                for f0 in range(0, D, LANES):
                    for d0 in range(0, LANES, 8):
                        vals = []
                        for d in range(d0, d0 + 8):
                            fcol = f0 + ((iot + d) & 15)
                            vals.append(
                                (fcol, plsc.load_gather(rows, [irow, fcol])))
                        for fcol, v in vals:
                            plsc.store_scatter(tb, [fcol, irow], v)
                return c

            lax.fori_loop(0, G // LANES, blk, 0)

        def writeback(g, tb, osem):
            for tr in range(tiles_f):
                pltpu.async_copy(tb.at[pl.ds(tr * 8, 8)],
                                 out_hbm.at[tr, gbase + g], osem)

        def drain_out(tb, osem):
            for tr in range(tiles_f):
                pltpu.make_async_copy(tb.at[pl.ds(tr * 8, 8)],
                                      out_hbm.at[tr, 0], osem).wait()

        gather(0, rows0, gsem0).start()

        def pair_body(p, carry):
            g0 = 2 * p
            gather(g0 + 1, rows1, gsem1).start()
            gather(g0, rows0, gsem0).wait()

            @pl.when(p > 0)
            def _():
                drain_out(tb0, osem0)

            transpose(rows0, tb0)
            writeback(g0, tb0, osem0)

            @pl.when(p < pairs - 1)
            def _():
                gather(g0 + 2, rows0, gsem0).start()

            gather(g0 + 1, rows1, gsem1).wait()

            @pl.when(p > 0)
            def _():
                drain_out(tb1, osem1)

            transpose(rows1, tb1)
            writeback(g0 + 1, tb1, osem1)
            return carry

        lax.fori_loop(0, pairs, pair_body, 0)
        drain_out(tb0, osem0)
        drain_out(tb1, osem1)

    return emb


def kernel(x, weight):
    B = x.size
    V, D = weight.shape
    idx = x.reshape(B // G, G).astype(jnp.int32)
    wpad = jnp.pad(weight, ((0, 0), (0, WPAD - D)))
    emb = _make_kernel(B, V, D)
    out4 = emb(wpad, idx)
    out_t = out4.transpose(0, 2, 1, 3).reshape(D, B)
    return out_t.T.reshape(B, 1, D)


# speedup vs baseline: 1.3750x; 1.0127x over previous
import functools

import jax
import jax.numpy as jnp
from jax import lax
from jax.experimental import pallas as pl
from jax.experimental.pallas import tpu as pltpu
from jax.experimental.pallas import tpu_sc as plsc

NUM_CORES = 2
NUM_SUBCORES = 16
NUM_WORKERS = NUM_CORES * NUM_SUBCORES

LANES = 16
G = 128


WPAD = 128


@functools.lru_cache(maxsize=None)
def _make_kernel(B, V, D):
    assert D == 32 and B % (NUM_WORKERS * 2 * G) == 0
    b_per_w = B // NUM_WORKERS
    groups_pw = b_per_w // G
    pairs = groups_pw // 2
    tiles_f = D // 8

    mesh = plsc.VectorSubcoreMesh(core_axis_name="c", subcore_axis_name="s")

    @functools.partial(
        pl.kernel,
        mesh=mesh,
        compiler_params=pltpu.CompilerParams(use_tc_tiling_on_sc=False,
                                             needs_layout_passes=False),
        out_type=jax.ShapeDtypeStruct((tiles_f, B // G, 8, G), jnp.float32),
        scratch_types=[
            pltpu.VMEM((groups_pw, G), jnp.int32),
            pltpu.VMEM((G, WPAD), jnp.float32),
            pltpu.VMEM((G, WPAD), jnp.float32),
            pltpu.VMEM((D, G), jnp.float32),
            pltpu.VMEM((D, G), jnp.float32),
            pltpu.SemaphoreType.DMA,
            pltpu.SemaphoreType.DMA,
            pltpu.SemaphoreType.DMA,
            pltpu.SemaphoreType.DMA,
        ],
    )
    def emb(table_hbm, idx_hbm, out_hbm, idx_v, rows0, rows1, tb0, tb1,
            gsem0, gsem1, osem0, osem1):
        wid = lax.axis_index("s") * NUM_CORES + lax.axis_index("c")
        gbase = wid * groups_pw
        pltpu.sync_copy(idx_hbm.at[pl.ds(gbase, groups_pw)], idx_v)

        def gather(g, rows, gsem):
            return pltpu.make_async_copy(table_hbm.at[idx_v.at[g]], rows, gsem)

        def transpose(rows, tb):
            iot = lax.iota(jnp.int32, 16)

            def blk(bi, c):
                irow = bi * LANES + iot
                for f0 in range(0, D, LANES):
                    vals = []
                    for d in range(LANES):
                        fcol = f0 + ((iot + d) & 15)
                        vals.append(
                            (fcol, plsc.load_gather(rows, [irow, fcol])))
                    for fcol, v in vals:
                        plsc.store_scatter(tb, [fcol, irow], v)
                return c

            lax.fori_loop(0, G // LANES, blk, 0)

        def writeback(g, tb, osem):
            for tr in range(tiles_f):
                pltpu.async_copy(tb.at[pl.ds(tr * 8, 8)],
                                 out_hbm.at[tr, gbase + g], osem)

        def drain_out(tb, osem):
            for tr in range(tiles_f):
                pltpu.make_async_copy(tb.at[pl.ds(tr * 8, 8)],
                                      out_hbm.at[tr, 0], osem).wait()

        gather(0, rows0, gsem0).start()

        def pair_body(p, carry):
            g0 = 2 * p
            gather(g0 + 1, rows1, gsem1).start()
            gather(g0, rows0, gsem0).wait()

            @pl.when(p > 0)
            def _():
                drain_out(tb0, osem0)

            transpose(rows0, tb0)
            writeback(g0, tb0, osem0)

            @pl.when(p < pairs - 1)
            def _():
                gather(g0 + 2, rows0, gsem0).start()

            gather(g0 + 1, rows1, gsem1).wait()

            @pl.when(p > 0)
            def _():
                drain_out(tb1, osem1)

            transpose(rows1, tb1)
            writeback(g0 + 1, tb1, osem1)
            return carry

        lax.fori_loop(0, pairs, pair_body, 0)
        drain_out(tb0, osem0)
        drain_out(tb1, osem1)

    return emb


def kernel(x, weight):
    B = x.size
    V, D = weight.shape
    idx = x.reshape(B // G, G).astype(jnp.int32)
    wpad = jnp.pad(weight, ((0, 0), (0, WPAD - D)))
    emb = _make_kernel(B, V, D)
    out4 = emb(wpad, idx)
    out_t = out4.transpose(0, 2, 1, 3).reshape(D, B)
    return out_t.T.reshape(B, 1, D)
